# Initial kernel scaffold; baseline (speedup 1.0000x reference)
import jax
import jax.numpy as jnp
from jax.experimental import pallas as pl


def kernel(x, edge_index, edge_type, W1, root1, b1, W2, root2, b2, Wc, bc):
    raise NotImplementedError("write your pallas kernel here")


# speedup vs baseline: 6.5447x; 6.5447x over previous
import jax
import jax.numpy as jnp
from jax import lax
from jax.experimental import pallas as pl
from jax.experimental.pallas import tpu as pltpu
from jax.experimental.pallas import tpu_sc as plsc

N = 10000
E = 320000
R = 8
HID = 128
HHID = HID // 2
CLS = 16

NC = 2
NS = 16
NW = NC * NS
EPW = E // NW
C = 80
NCH = EPW // C
NPAIR = NCH // 2
PB = (0, 2504, 5008, 7512, 10000)
PMAX = 2504
ZS = 152
NR = N * R
BIGC = 2000
NB = 25
BN = N // NB

_MESH = plsc.VectorSubcoreMesh(core_axis_name="c", subcore_axis_name="s")
_SC_PARAMS = pltpu.CompilerParams(needs_layout_passes=False)


NWREP = NC * NS
EPW = E // NWREP


def _prep1_body(src_h, dst_h, et_h, zc2_h, gidx_h, key2_h, cntp_h,
                src_v, dst_v, et_v, gx_v, k2_v, cnt2d_v):
    cid = lax.axis_index("c")
    sid = lax.axis_index("s")
    wid = cid * NS + sid
    pltpu.sync_copy(zc2_h, cnt2d_v)
    ones16 = jnp.full((16,), 1.0, jnp.float32)
    for b in range(EPW // BIGC):
        eb = wid * EPW + b * BIGC
        pltpu.sync_copy(src_h.at[pl.ds(eb, BIGC)], src_v)
        pltpu.sync_copy(dst_h.at[pl.ds(eb, BIGC)], dst_v)
        pltpu.sync_copy(et_h.at[pl.ds(eb, BIGC)], et_v)

        def cbody(j, carry):
            sl = pl.ds(j * 16, 16)
            s = src_v[sl]
            e = et_v[sl]
            d = dst_v[sl]
            gx_v[sl] = e * N + s
            k2 = d * R + e
            k2_v[sl] = k2
            plsc.addupdate_scatter(cnt2d_v, [k2 >> 7, k2 & 127], ones16)
            return carry

        lax.fori_loop(0, BIGC // 16, cbody, 0)
        pltpu.sync_copy(gx_v, gidx_h.at[pl.ds(eb, BIGC)])
        pltpu.sync_copy(k2_v, key2_h.at[pl.ds(eb, BIGC)])

    pltpu.sync_copy(cnt2d_v, cntp_h.at[wid])


_prep1 = pl.kernel(
    _prep1_body,
    out_type=(jax.ShapeDtypeStruct((E,), jnp.int32),
              jax.ShapeDtypeStruct((E,), jnp.int32),
              jax.ShapeDtypeStruct((NWREP, NR // 128, 128), jnp.float32)),
    mesh=_MESH,
    scratch_types=[
        pltpu.VMEM((BIGC,), jnp.int32),
        pltpu.VMEM((BIGC,), jnp.int32),
        pltpu.VMEM((BIGC,), jnp.int32),
        pltpu.VMEM((BIGC,), jnp.int32),
        pltpu.VMEM((BIGC,), jnp.int32),
        pltpu.VMEM((NR // 128, 128), jnp.float32),
    ],
    compiler_params=_SC_PARAMS,
)


def _cs_body(c_ref, inv_ref):
    s = jnp.sum(c_ref[...], axis=0)
    inv_ref[...] = 1.0 / jnp.maximum(s, 1.0)


def _cs_call(cntp):
    return pl.pallas_call(
        _cs_body,
        out_shape=jax.ShapeDtypeStruct((NR // 128, 128), jnp.float32),
    )(cntp)


def _prep2_body(key2_h, inv_h, norm_h, k2b_v, inv_v, normb_v):
    cid = lax.axis_index("c")
    sid = lax.axis_index("s")
    wid = cid * NS + sid
    pltpu.sync_copy(inv_h, inv_v)
    pltpu.sync_copy(key2_h.at[pl.ds(wid * EPW, EPW)], k2b_v)

    def nb(j, carry):
        k2 = k2b_v[pl.ds(j * 16, 16)]
        normb_v[pl.ds(j * 16, 16)] = plsc.load_gather(inv_v, [k2])
        return carry

    lax.fori_loop(0, EPW // 16, nb, 0)
    pltpu.sync_copy(normb_v, norm_h.at[pl.ds(wid * EPW, EPW)])


_prep2 = pl.kernel(
    _prep2_body,
    out_type=jax.ShapeDtypeStruct((E,), jnp.float32),
    mesh=_MESH,
    scratch_types=[
        pltpu.VMEM((EPW,), jnp.int32),
        pltpu.VMEM((NR,), jnp.float32),
        pltpu.VMEM((EPW,), jnp.float32),
    ],
    compiler_params=_SC_PARAMS,
)


def _msk_body(d_ref, n_ref, dm_ref, nm_ref):
    d = d_ref[...]
    n = n_ref[...]
    for p in range(4):
        lo, hi = PB[p], PB[p + 1]
        m = (d >= lo) & (d < hi)
        nm_ref[p] = jnp.where(m, n, 0.0)
        dm_ref[p] = jnp.clip(d - lo, 0, hi - lo - 1)


def _msk_call(dst2, norm2):
    return pl.pallas_call(
        _msk_body,
        out_shape=[
            jax.ShapeDtypeStruct((4, E // 128, 128), jnp.int32),
            jax.ShapeDtypeStruct((4, E // 128, 128), jnp.float32),
        ],
    )(dst2, norm2)


def _agg_body(y_h, gidx_h, dm_h, nm_h, zr_h, part_h,
              g0, g1, d0, d1, n0, n1, sdt, r0, r1, h0, h1,
              semA0, semA1, semR0, semR1, semS0, semS1, zr_v, agg_sh):
    cid = lax.axis_index("c")
    sid = lax.axis_index("s")
    ebase = (cid * NS + sid) * EPW

    def waitA(gv, dv, nv, sem):
        pltpu.make_async_copy(gidx_h.at[pl.ds(0, C)], gv, sem).wait()
        pltpu.make_async_copy(dm_h.at[pl.ds(0, C)], dv, sem).wait()
        pltpu.make_async_copy(nm_h.at[pl.ds(0, C)], nv, sem).wait()

    def issueR(gv, rv, sem):
        pltpu.async_copy(y_h.at[gv], rv, sem)

    def waitR(gv, rv, sem):
        pltpu.make_async_copy(y_h.at[gv], rv, sem).wait()

    def issueS(hv, slot, sem):
        pltpu.async_copy(hv, agg_sh.at[sdt.at[slot]], sem, add=True)

    def waitS(hv, slot, sem):
        pltpu.make_async_copy(hv, agg_sh.at[sdt.at[slot]], sem).wait()

    for p in range(4):
        lo = PB[p]
        rows = PB[p + 1] - lo
        tail = rows - NS * ZS

        def loadA(g, gv, dv, nv, sem):
            base = ebase + g * C
            moff = p * E + base
            pltpu.async_copy(gidx_h.at[pl.ds(base, C)], gv, sem)
            pltpu.async_copy(dm_h.at[pl.ds(moff, C)], dv, sem)
            pltpu.async_copy(nm_h.at[pl.ds(moff, C)], nv, sem)

        pltpu.sync_copy(zr_h, zr_v)
        pltpu.sync_copy(zr_v, agg_sh.at[pl.ds(sid * ZS, ZS)])

        @pl.when(sid == 0)
        def _():
            pltpu.sync_copy(zr_v.at[pl.ds(0, tail)],
                            agg_sh.at[pl.ds(NS * ZS, tail)])

        plsc.subcore_barrier()

        def compute(dv, nv, slot, rv, hv):
            for u in range(C // 16):
                sl = pl.ds(u * 16, 16)
                sdt[slot, sl] = dv[sl]

            def sc_body(k, carry):
                sp = plsc.load_gather(nv, [jnp.full((16,), k, jnp.int32)])
                for cc in range(HID // 16):
                    hv[k, pl.ds(cc * 16, 16)] = (
                        rv[k, pl.ds(cc * 16, 16)] * sp)
                return carry

            lax.fori_loop(0, C, sc_body, 0)

        loadA(0, g0, d0, n0, semA0)
        waitA(g0, d0, n0, semA0)
        issueR(g0, r0, semR0)
        loadA(1, g1, d1, n1, semA1)

        def pair(t, carry):
            waitA(g1, d1, n1, semA1)
            issueR(g1, r1, semR1)

            @pl.when(t >= 1)
            def _():
                waitS(h0, 0, semS0)

            waitR(g0, r0, semR0)
            compute(d0, n0, 0, r0, h0)
            issueS(h0, 0, semS0)
            loadA(2 * t + 2, g0, d0, n0, semA0)

            waitA(g0, d0, n0, semA0)
            issueR(g0, r0, semR0)

            @pl.when(t >= 1)
            def _():
                waitS(h1, 1, semS1)

            waitR(g1, r1, semR1)
            compute(d1, n1, 1, r1, h1)
            issueS(h1, 1, semS1)

            @pl.when(t < NPAIR - 1)
            def _():
                loadA(2 * t + 3, g1, d1, n1, semA1)

            return carry

        lax.fori_loop(0, NPAIR, pair, 0)
        waitS(h0, 0, semS0)
        waitR(g0, r0, semR0)
        compute(d0, n0, 0, r0, h0)
        issueS(h0, 0, semS0)
        waitS(h0, 0, semS0)
        waitS(h1, 1, semS1)
        plsc.subcore_barrier()

        pltpu.sync_copy(agg_sh.at[pl.ds(sid * ZS, ZS)], zr_v)
        pltpu.sync_copy(zr_v, part_h.at[cid, pl.ds(lo + sid * ZS, ZS)])

        @pl.when(sid == 0)
        def _():
            pltpu.sync_copy(agg_sh.at[pl.ds(NS * ZS, tail)],
                            zr_v.at[pl.ds(0, tail)])
            pltpu.sync_copy(zr_v.at[pl.ds(0, tail)],
                            part_h.at[cid, pl.ds(lo + NS * ZS, tail)])

        plsc.subcore_barrier()


_agg = pl.kernel(
    _agg_body,
    out_type=jax.ShapeDtypeStruct((NC, N, HID), jnp.float32),
    mesh=_MESH,
    scratch_types=[
        pltpu.VMEM((C,), jnp.int32),
        pltpu.VMEM((C,), jnp.int32),
        pltpu.VMEM((C,), jnp.int32),
        pltpu.VMEM((C,), jnp.int32),
        pltpu.VMEM((C,), jnp.float32),
        pltpu.VMEM((C,), jnp.float32),
        pltpu.VMEM((2, C), jnp.int32),
        pltpu.VMEM((C, HID), jnp.float32),
        pltpu.VMEM((C, HID), jnp.float32),
        pltpu.VMEM((C, HID), jnp.float32),
        pltpu.VMEM((C, HID), jnp.float32),
        pltpu.SemaphoreType.DMA,
        pltpu.SemaphoreType.DMA,
        pltpu.SemaphoreType.DMA,
        pltpu.SemaphoreType.DMA,
        pltpu.SemaphoreType.DMA,
        pltpu.SemaphoreType.DMA,
        pltpu.VMEM((ZS, HID), jnp.float32),
        pltpu.VMEM_SHARED((PMAX, HID), jnp.float32),
    ],
    compiler_params=_SC_PARAMS,
)


def _mm_body(x_ref, w_ref, b_ref, y_ref):
    r = pl.program_id(1)
    y = jnp.dot(x_ref[...], w_ref[0], preferred_element_type=jnp.float32)
    y_ref[...] = jnp.where(r == R, y + b_ref[...], y)


def _mm_call(x, we, b):
    return pl.pallas_call(
        _mm_body,
        grid=(NB, R + 1),
        in_specs=[
            pl.BlockSpec((BN, HID), lambda bi, r: (bi, 0)),
            pl.BlockSpec((1, HID, HID), lambda bi, r: (r, 0, 0)),
            pl.BlockSpec((1, HID), lambda bi, r: (0, 0)),
        ],
        out_specs=pl.BlockSpec((BN, HID), lambda bi, r: (r * NB + bi, 0)),
        out_shape=jax.ShapeDtypeStruct(((R + 1) * N, HID), jnp.float32),
    )(x, we, b)


def _hy2_body(part_ref, xr_ref, w_ref, y_ref, h_ref):
    r = pl.program_id(1)

    @pl.when(r == 0)
    def _():
        h_ref[...] = jnp.maximum(part_ref[0] + part_ref[1] + xr_ref[...],
                                 0.0)

    y_ref[...] = jnp.dot(h_ref[...], w_ref[0],
                         preferred_element_type=jnp.float32)


def _hy2_call(part1, y1, we):
    return pl.pallas_call(
        _hy2_body,
        grid=(NB, R + 1),
        in_specs=[
            pl.BlockSpec((NC, BN, HID), lambda bi, r: (0, bi, 0)),
            pl.BlockSpec((BN, HID), lambda bi, r: (R * NB + bi, 0)),
            pl.BlockSpec((1, HID, HID), lambda bi, r: (r, 0, 0)),
        ],
        out_specs=pl.BlockSpec((BN, HID), lambda bi, r: (r * NB + bi, 0)),
        out_shape=jax.ShapeDtypeStruct(((R + 1) * N, HID), jnp.float32),
        scratch_shapes=[pltpu.VMEM((BN, HID), jnp.float32)],
    )(part1, y1, we)


def _fin_body(part_ref, hr_ref, b2_ref, wc_ref, bc_ref, out_ref, emb_ref):
    emb = part_ref[0] + part_ref[1] + hr_ref[...] + b2_ref[...]
    emb_ref[...] = emb
    out_ref[...] = jnp.dot(emb, wc_ref[...],
                           preferred_element_type=jnp.float32) + bc_ref[...]


def _fin_call(part2, y2, b2, wc, bc):
    return pl.pallas_call(
        _fin_body,
        grid=(NB,),
        in_specs=[
            pl.BlockSpec((NC, BN, HID), lambda bi: (0, bi, 0)),
            pl.BlockSpec((BN, HID), lambda bi: (R * NB + bi, 0)),
            pl.BlockSpec((1, HID), lambda bi: (0, 0)),
            pl.BlockSpec((HID, CLS), lambda bi: (0, 0)),
            pl.BlockSpec((1, CLS), lambda bi: (0, 0)),
        ],
        out_specs=[
            pl.BlockSpec((BN, CLS), lambda bi: (bi, 0)),
            pl.BlockSpec((BN, HID), lambda bi: (bi, 0)),
        ],
        out_shape=[
            jax.ShapeDtypeStruct((N, CLS), jnp.float32),
            jax.ShapeDtypeStruct((N, HID), jnp.float32),
        ],
    )(part2, y2, b2, wc, bc)


def kernel(x, edge_index, edge_type, W1, root1, b1, W2, root2, b2, Wc, bc):
    src = edge_index[0]
    dst = edge_index[1]
    zc2 = jnp.zeros((NR // 128, 128), jnp.float32)
    zr = jnp.zeros((ZS, HID), jnp.float32)

    gidx, key2e, cntp = _prep1(src, dst, edge_type, zc2)
    inv = _cs_call(cntp).reshape(NR)
    norm_e = _prep2(key2e, inv)
    dm, nm = _msk_call(dst.reshape(E // 128, 128),
                       norm_e.reshape(E // 128, 128))
    dm, nm = dm.reshape(4 * E), nm.reshape(4 * E)

    w1e = jnp.concatenate([W1, root1[None]], axis=0)
    y1 = _mm_call(x, w1e, b1.reshape(1, HID))
    part1 = _agg(y1, gidx, dm, nm, zr)

    w2e = jnp.concatenate([W2, root2[None]], axis=0)
    y2 = _hy2_call(part1, y1, w2e)
    part2 = _agg(y2, gidx, dm, nm, zr)

    out, emb = _fin_call(part2, y2, b2.reshape(1, HID), Wc, bc.reshape(1, CLS))
    return (out, emb)
